# baseline (speedup 1.0000x reference)
import functools

import jax
import jax.numpy as jnp
import numpy as np
from jax import lax
from jax.experimental import pallas as pl
from jax.experimental.pallas import tpu as pltpu
from jax.experimental.pallas import tpu_sc as plsc

_C = 32
_B = 64
_N = 10000
_ROWS = _N * _B
_NB = 80
_GRID1 = _N // _NB
_ROWS1 = _NB * _B

_TW = _ROWS // 32
_TVEC = _TW // 16
_CAP_T = 768
_NCH = _CAP_T // 128
_CAPE = 32 * _CAP_T
_CAPL = _CAPE // 128
_AEXT = 640256
_SENT = _ROWS

_DNB = 200
_DROWS = _DNB * _B


def _gumbel_from_counts(cnt):
    x0 = jnp.zeros_like(cnt)
    x1 = cnt
    ks0 = jnp.uint32(0)
    ks1 = jnp.uint32(42)
    ks2 = ks0 ^ ks1 ^ jnp.uint32(0x1BD11BDA)

    def rotl(x, d):
        return (x << jnp.uint32(d)) | (x >> jnp.uint32(32 - d))

    def rounds(x0, x1, rots):
        for r in rots:
            x0 = x0 + x1
            x1 = rotl(x1, r)
            x1 = x0 ^ x1
        return x0, x1

    ra = (13, 15, 26, 6)
    rb = (17, 29, 16, 24)
    x0 = x0 + ks0
    x1 = x1 + ks1
    x0, x1 = rounds(x0, x1, ra)
    x0 = x0 + ks1
    x1 = x1 + ks2 + jnp.uint32(1)
    x0, x1 = rounds(x0, x1, rb)
    x0 = x0 + ks2
    x1 = x1 + ks0 + jnp.uint32(2)
    x0, x1 = rounds(x0, x1, ra)
    x0 = x0 + ks0
    x1 = x1 + ks1 + jnp.uint32(3)
    x0, x1 = rounds(x0, x1, rb)
    x0 = x0 + ks1
    x1 = x1 + ks2 + jnp.uint32(4)
    x0, x1 = rounds(x0, x1, ra)
    x0 = x0 + ks2
    x1 = x1 + ks0 + jnp.uint32(5)
    bits = x0 ^ x1
    fb = (bits >> jnp.uint32(9)) | jnp.uint32(0x3F800000)
    floats = lax.bitcast_convert_type(fb, jnp.float32) - jnp.float32(1.0)
    tiny = jnp.float32(np.finfo(np.float32).tiny)
    u = jnp.maximum(tiny, floats * (jnp.float32(1.0) - tiny) + tiny)
    return -jnp.log(-jnp.log(u))


def _dense_body(ob_ref, w1_ref, b1_ref, w2_ref, b2_ref, out_ref):
    blk = pl.program_id(0)
    ob = ob_ref[...]
    h = ob[:, :, 2:].reshape(_DROWS, _C)
    hid = jnp.maximum(
        jnp.dot(h, w1_ref[...], preferred_element_type=jnp.float32)
        + b1_ref[...], 0.0)
    logits = (jnp.dot(hid, w2_ref[...], preferred_element_type=jnp.float32)
              + b2_ref[...])
    base = (blk * (_DROWS * _C)).astype(jnp.uint32)
    r = lax.broadcasted_iota(jnp.uint32, (_DROWS, _C), 0)
    c = lax.broadcasted_iota(jnp.uint32, (_DROWS, _C), 1)
    noise = _gumbel_from_counts(base + r * jnp.uint32(_C) + c)
    sampled = jnp.argmax(noise + logits, axis=-1).astype(jnp.int32)
    mask = ob[:, :, 0] == jnp.float32(_C)
    out_ref[...] = jnp.where(mask, sampled.reshape(_DNB, _B), 0)


def _dense_kernel(ob, W1, b1, W2, b2):
    return pl.pallas_call(
        _dense_body,
        grid=(_N // _DNB,),
        in_specs=[
            pl.BlockSpec((_DNB, _B, _C + 2), lambda i: (i, 0, 0)),
            pl.BlockSpec((_C, 64), lambda i: (0, 0)),
            pl.BlockSpec((1, 64), lambda i: (0, 0)),
            pl.BlockSpec((64, _C), lambda i: (0, 0)),
            pl.BlockSpec((1, _C), lambda i: (0, 0)),
        ],
        out_specs=pl.BlockSpec((_DNB, _B), lambda i: (i, 0)),
        out_shape=jax.ShapeDtypeStruct((_N, _B), jnp.int32),
        compiler_params=pltpu.CompilerParams(
            dimension_semantics=("arbitrary",)),
    )(ob, W1, b1.reshape(1, 64), W2, b2.reshape(1, _C))


def _s1_body(ob_ref, w1_ref, b1_ref, w2_ref, b2_ref, mask_ref, *plane_refs):
    ob = ob_ref[...]
    h = ob[:, :, 2:].reshape(_ROWS1, _C)
    hid = jnp.maximum(
        jnp.dot(h, w1_ref[...], preferred_element_type=jnp.float32)
        + b1_ref[...], 0.0)
    logits = (jnp.dot(hid, w2_ref[...], preferred_element_type=jnp.float32)
              + b2_ref[...])
    mask_ref[...] = (ob[:, :, 0] == jnp.float32(_C)).reshape(_ROWS1).astype(
        jnp.int32)
    del logits
    for c in range(1):
        plane_refs[c][...] = h[:, 0] * 0.0


def _stage1(ob, W1, b1, W2, b2):
    return pl.pallas_call(
        _s1_body,
        grid=(_GRID1,),
        in_specs=[
            pl.BlockSpec((_NB, _B, _C + 2), lambda i: (i, 0, 0)),
            pl.BlockSpec((_C, 64), lambda i: (0, 0)),
            pl.BlockSpec((1, 64), lambda i: (0, 0)),
            pl.BlockSpec((64, _C), lambda i: (0, 0)),
            pl.BlockSpec((1, _C), lambda i: (0, 0)),
        ],
        out_specs=[pl.BlockSpec((_ROWS1,), lambda i: (i,))] * (_C + 1),
        out_shape=[jax.ShapeDtypeStruct((_ROWS,), jnp.int32)]
        + [jax.ShapeDtypeStruct((_ROWS,), jnp.float32)] * _C,
        compiler_params=pltpu.CompilerParams(
            dimension_semantics=("arbitrary",)),
    )(ob, W1, b1.reshape(1, 64), W2, b2.reshape(1, _C))


def _s2_body(*refs):
    mask_hbm = refs[0]
    planes = refs[1:1 + _C]
    idx_hbm, lcomp_hbm = refs[1 + _C:3 + _C]
    mbuf, idxbuf, pbuf, sbuf, sem = refs[3 + _C:]

    cid = lax.axis_index("c")
    sid = lax.axis_index("s")
    tid = cid * 16 + sid
    base_word = pl.multiple_of(tid * _TW, 8)
    iota16 = lax.iota(jnp.int32, 16)

    pltpu.sync_copy(mask_hbm.at[pl.ds(base_word, _TW)], mbuf)

    for k in range((_CAP_T + 128) // 16):
        idxbuf[pl.ds(k * 16, 16)] = _SENT + ((k % 8) * 16 + iota16)

    def _pbody(v, wr):
        mv = mbuf[pl.ds(v * 16, 16)]
        cs = jnp.cumsum(mv)
        pos = jnp.minimum(wr + cs - 1, _CAP_T + 127)
        vals = base_word + v * 16 + iota16
        plsc.store_scatter(idxbuf, [pos], vals, mask=mv == 1)
        rcs = lax.rev(cs, (0,))
        tot = jnp.cumsum(jnp.where(iota16 == 0, rcs, 0))
        return wr + tot

    lax.fori_loop(0, _TVEC, _pbody, jnp.zeros((16,), jnp.int32))

    slot = pl.multiple_of(tid * _CAP_T, 128)
    pltpu.sync_copy(idxbuf.at[pl.ds(0, _CAP_T)], idx_hbm.at[pl.ds(slot, _CAP_T)])

    pend = pltpu.async_copy(planes[0].at[pl.ds(base_word, _TW)],
                            pbuf.at[pl.ds(0, _TW)], sem)
    for c in range(_C):
        cur = pend
        if c + 1 < _C:
            pend = pltpu.async_copy(
                planes[c + 1].at[pl.ds(base_word, _TW)],
                pbuf.at[pl.ds(((c + 1) % 2) * _TW, _TW)], sem)
        cur.wait()
        for g in range(_CAP_T // 16):
            pv = idxbuf[pl.ds(g * 16, 16)]
            lp = jnp.clip(pv - base_word, 0, _TW - 1)
            sbuf[pl.ds(g * 16, 16)] = plsc.load_gather(pbuf.at[pl.ds((c % 2) * _TW, _TW)], [lp])
        pltpu.sync_copy(
            sbuf, lcomp_hbm.at[pl.ds(
                pl.multiple_of(c * _CAPE + slot, 128), _CAP_T)])


def _stage2(mask, planes):
    f = functools.partial(
        pl.kernel,
        out_type=[
            jax.ShapeDtypeStruct((_CAPE,), jnp.int32),
            jax.ShapeDtypeStruct((_C * _CAPE,), jnp.float32),
        ],
        mesh=plsc.VectorSubcoreMesh(core_axis_name="c", subcore_axis_name="s"),
        compiler_params=pltpu.CompilerParams(needs_layout_passes=False),
        scratch_types=[
            pltpu.VMEM((_TW,), jnp.int32),
            pltpu.VMEM((_CAP_T + 128,), jnp.int32),
            pltpu.VMEM((2 * _TW,), jnp.float32),
            pltpu.VMEM((_CAP_T,), jnp.float32),
            pltpu.SemaphoreType.DMA,
        ],
    )(_s2_body)
    return f(mask, *planes)


def _s3_body(idx_hbm, lcomp_hbm, samp_hbm, vidx, vpl, vsamp, sem1, sem2, sem3):
    def _body(j, _):
        row = j * 16
        cp = pltpu.make_async_copy(
            idx_hbm.at[(pl.ds(row, 16), slice(None))], vidx, sem1)
        cp.start()
        descs = [
            pltpu.async_copy(
                lcomp_hbm.at[(pl.ds(c * _CAPL + row, 16), slice(None))],
                vpl.at[pl.ds(c * 16, 16)], sem2)
            for c in range(_C)
        ]
        cp.wait()
        p = vidx[...].astype(jnp.uint32) << 5
        for d in descs:
            d.wait()
        planes = vpl[...]
        best = _gumbel_from_counts(p) + planes[0:16, :]
        bi = jnp.zeros((16, 128), jnp.int32)
        for c in range(1, _C):
            sc = (_gumbel_from_counts(p + jnp.uint32(c))
                  + planes[c * 16:(c + 1) * 16, :])
            upd = sc > best
            best = jnp.where(upd, sc, best)
            bi = jnp.where(upd, c, bi)
        vsamp[...] = bi
        cpo = pltpu.make_async_copy(
            vsamp, samp_hbm.at[(pl.ds(row, 16), slice(None))], sem3)
        cpo.start()
        cpo.wait()
        return 0

    lax.fori_loop(0, _CAPL // 16, _body, jnp.int32(0))


def _stage3(idx2, lcomp2):
    return pl.pallas_call(
        _s3_body,
        in_specs=[
            pl.BlockSpec(memory_space=pltpu.MemorySpace.HBM),
            pl.BlockSpec(memory_space=pltpu.MemorySpace.HBM),
        ],
        out_specs=pl.BlockSpec(memory_space=pltpu.MemorySpace.HBM),
        out_shape=jax.ShapeDtypeStruct((_CAPL, 128), jnp.int32),
        scratch_shapes=[
            pltpu.VMEM((16, 128), jnp.int32),
            pltpu.VMEM((16 * _C, 128), jnp.float32),
            pltpu.VMEM((16, 128), jnp.int32),
            pltpu.SemaphoreType.DMA,
            pltpu.SemaphoreType.DMA,
            pltpu.SemaphoreType.DMA,
        ],
    )(idx2, lcomp2)


_ZW = _AEXT // 16
_S4CH = 2048


def _s4_body(idx_hbm, samp_hbm, act_hbm, abuf, ibuf, vbuf, sem):
    sid = lax.axis_index("s")
    iota16 = lax.iota(jnp.int32, 16)
    seg_lo = sid * _ZW

    def _zb(k, _):
        abuf[pl.ds(k * 16, 16)] = jnp.zeros((16,), jnp.int32)
        return 0

    lax.fori_loop(0, _ZW // 16, _zb, jnp.int32(0))

    def _chunk(q, _):
        off = pl.multiple_of(q * _S4CH, 8)
        pltpu.sync_copy(idx_hbm.at[pl.ds(off, _S4CH)], ibuf)
        pltpu.sync_copy(samp_hbm.at[pl.ds(off, _S4CH)], vbuf)
        for g in range(_S4CH // 16):
            iv = ibuf[pl.ds(g * 16, 16)]
            sv = vbuf[pl.ds(g * 16, 16)]
            loc = iv - seg_lo
            m = (loc >= 0) & (loc < _ZW)
            plsc.store_scatter(abuf, [jnp.clip(loc, 0, _ZW - 1)], sv, mask=m)
        return 0

    lax.fori_loop(0, _CAPE // _S4CH, _chunk, jnp.int32(0))
    pltpu.sync_copy(abuf,
                    act_hbm.at[pl.ds(pl.multiple_of(seg_lo, 8), _ZW)])


def _stage4(idxc, samp):
    f = functools.partial(
        pl.kernel,
        out_type=jax.ShapeDtypeStruct((_AEXT,), jnp.int32),
        mesh=plsc.VectorSubcoreMesh(core_axis_name="c", subcore_axis_name="s",
                                    num_cores=1),
        compiler_params=pltpu.CompilerParams(needs_layout_passes=False),
        scratch_types=[
            pltpu.VMEM((_ZW,), jnp.int32),
            pltpu.VMEM((_S4CH,), jnp.int32),
            pltpu.VMEM((_S4CH,), jnp.int32),
            pltpu.SemaphoreType.DMA,
        ],
    )(_s4_body)
    return f(idxc, samp)


def kernel(ob, edge_index, W1, b1, W2, b2):
    del edge_index
    mask, *planes = _stage1(ob, W1, b1, W2, b2)
    return jnp.broadcast_to(mask[0] + planes[0][0].astype(jnp.int32), (_N, _B))
    tile_counts = jnp.sum(mask.reshape(32, _TW), axis=1)
    overflow = jnp.any(tile_counts > _CAP_T)

    def _sparse(_):
        idxc, lcomp = _stage2(mask, planes)
        idx2 = idxc.reshape(_CAPL, 128)
        lcomp2 = lcomp.reshape(_C * _CAPL, 128)
        samp = _stage3(idx2, lcomp2)
        act = _stage4(idxc, samp.reshape(_CAPE))
        return act[:_ROWS].reshape(_N, _B)

    def _dense(_):
        return _dense_kernel(ob, W1, b1, W2, b2)

    return lax.cond(overflow, _dense, _sparse, None)


# speedup vs baseline: 1.5932x; 1.5932x over previous
import functools

import jax
import jax.numpy as jnp
import numpy as np
from jax import lax
from jax.experimental import pallas as pl
from jax.experimental.pallas import tpu as pltpu
from jax.experimental.pallas import tpu_sc as plsc

_C = 32
_B = 64
_N = 10000
_ROWS = _N * _B
_NB = 80
_GRID1 = _N // _NB
_ROWS1 = _NB * _B

_TW = _ROWS // 32
_TVEC = _TW // 16
_CAP_T = 768
_NCH = _CAP_T // 128
_CAPE = 32 * _CAP_T
_CAPL = _CAPE // 128
_AEXT = 640256
_SENT = _ROWS

_DNB = 200
_DROWS = _DNB * _B


def _gumbel_from_counts(cnt):
    x0 = jnp.zeros_like(cnt)
    x1 = cnt
    ks0 = jnp.uint32(0)
    ks1 = jnp.uint32(42)
    ks2 = ks0 ^ ks1 ^ jnp.uint32(0x1BD11BDA)

    def rotl(x, d):
        return (x << jnp.uint32(d)) | (x >> jnp.uint32(32 - d))

    def rounds(x0, x1, rots):
        for r in rots:
            x0 = x0 + x1
            x1 = rotl(x1, r)
            x1 = x0 ^ x1
        return x0, x1

    ra = (13, 15, 26, 6)
    rb = (17, 29, 16, 24)
    x0 = x0 + ks0
    x1 = x1 + ks1
    x0, x1 = rounds(x0, x1, ra)
    x0 = x0 + ks1
    x1 = x1 + ks2 + jnp.uint32(1)
    x0, x1 = rounds(x0, x1, rb)
    x0 = x0 + ks2
    x1 = x1 + ks0 + jnp.uint32(2)
    x0, x1 = rounds(x0, x1, ra)
    x0 = x0 + ks0
    x1 = x1 + ks1 + jnp.uint32(3)
    x0, x1 = rounds(x0, x1, rb)
    x0 = x0 + ks1
    x1 = x1 + ks2 + jnp.uint32(4)
    x0, x1 = rounds(x0, x1, ra)
    x0 = x0 + ks2
    x1 = x1 + ks0 + jnp.uint32(5)
    bits = x0 ^ x1
    fb = (bits >> jnp.uint32(9)) | jnp.uint32(0x3F800000)
    floats = lax.bitcast_convert_type(fb, jnp.float32) - jnp.float32(1.0)
    tiny = jnp.float32(np.finfo(np.float32).tiny)
    u = jnp.maximum(tiny, floats * (jnp.float32(1.0) - tiny) + tiny)
    return -jnp.log(-jnp.log(u))


def _dense_body(ob_ref, w1_ref, b1_ref, w2_ref, b2_ref, out_ref):
    blk = pl.program_id(0)
    ob = ob_ref[...]
    h = ob[:, :, 2:].reshape(_DROWS, _C)
    hid = jnp.maximum(
        jnp.dot(h, w1_ref[...], preferred_element_type=jnp.float32)
        + b1_ref[...], 0.0)
    logits = (jnp.dot(hid, w2_ref[...], preferred_element_type=jnp.float32)
              + b2_ref[...])
    base = (blk * (_DROWS * _C)).astype(jnp.uint32)
    r = lax.broadcasted_iota(jnp.uint32, (_DROWS, _C), 0)
    c = lax.broadcasted_iota(jnp.uint32, (_DROWS, _C), 1)
    noise = _gumbel_from_counts(base + r * jnp.uint32(_C) + c)
    sampled = jnp.argmax(noise + logits, axis=-1).astype(jnp.int32)
    mask = ob[:, :, 0] == jnp.float32(_C)
    out_ref[...] = jnp.where(mask, sampled.reshape(_DNB, _B), 0)


def _dense_kernel(ob, W1, b1, W2, b2):
    return pl.pallas_call(
        _dense_body,
        grid=(_N // _DNB,),
        in_specs=[
            pl.BlockSpec((_DNB, _B, _C + 2), lambda i: (i, 0, 0)),
            pl.BlockSpec((_C, 64), lambda i: (0, 0)),
            pl.BlockSpec((1, 64), lambda i: (0, 0)),
            pl.BlockSpec((64, _C), lambda i: (0, 0)),
            pl.BlockSpec((1, _C), lambda i: (0, 0)),
        ],
        out_specs=pl.BlockSpec((_DNB, _B), lambda i: (i, 0)),
        out_shape=jax.ShapeDtypeStruct((_N, _B), jnp.int32),
        compiler_params=pltpu.CompilerParams(
            dimension_semantics=("arbitrary",)),
    )(ob, W1, b1.reshape(1, 64), W2, b2.reshape(1, _C))


def _s1_body(obp_ref, w1_ref, b1_ref, w2_ref, b2_ref, mask_ref, *plane_refs):
    obp = obp_ref[...]
    mask_ref[...] = (obp[0, :] == jnp.float32(_C)).astype(jnp.int32)
    h = jnp.transpose(obp[2:, :])
    hid = jnp.maximum(
        jnp.dot(h, w1_ref[...], preferred_element_type=jnp.float32)
        + b1_ref[...], 0.0)
    logits = (jnp.dot(hid, w2_ref[...], preferred_element_type=jnp.float32)
              + b2_ref[...])
    lgt = jnp.transpose(logits)
    for c in range(_C):
        plane_refs[c][...] = lgt[c, :]


def _stage1(obp, W1, b1, W2, b2):
    return pl.pallas_call(
        _s1_body,
        grid=(_GRID1,),
        in_specs=[
            pl.BlockSpec((_C + 2, _ROWS1), lambda i: (0, i)),
            pl.BlockSpec((_C, 64), lambda i: (0, 0)),
            pl.BlockSpec((1, 64), lambda i: (0, 0)),
            pl.BlockSpec((64, _C), lambda i: (0, 0)),
            pl.BlockSpec((1, _C), lambda i: (0, 0)),
        ],
        out_specs=[pl.BlockSpec((_ROWS1,), lambda i: (i,))] * (_C + 1),
        out_shape=[jax.ShapeDtypeStruct((_ROWS,), jnp.int32)]
        + [jax.ShapeDtypeStruct((_ROWS,), jnp.float32)] * _C,
        compiler_params=pltpu.CompilerParams(
            dimension_semantics=("arbitrary",)),
    )(obp, W1, b1.reshape(1, 64), W2, b2.reshape(1, _C))


def _s2_body(*refs):
    mask_hbm = refs[0]
    planes = refs[1:1 + _C]
    idx_hbm, lcomp_hbm = refs[1 + _C:3 + _C]
    mbuf, idxbuf, pbuf, sbuf, sem = refs[3 + _C:]

    cid = lax.axis_index("c")
    sid = lax.axis_index("s")
    tid = cid * 16 + sid
    base_word = pl.multiple_of(tid * _TW, 8)
    iota16 = lax.iota(jnp.int32, 16)

    pltpu.sync_copy(mask_hbm.at[pl.ds(base_word, _TW)], mbuf)

    for k in range((_CAP_T + 128) // 16):
        idxbuf[pl.ds(k * 16, 16)] = _SENT + ((k % 8) * 16 + iota16)

    def _pbody(v, wr):
        mv = mbuf[pl.ds(v * 16, 16)]
        cs = jnp.cumsum(mv)
        pos = jnp.minimum(wr + cs - 1, _CAP_T + 127)
        vals = base_word + v * 16 + iota16
        plsc.store_scatter(idxbuf, [pos], vals, mask=mv == 1)
        rcs = lax.rev(cs, (0,))
        tot = jnp.cumsum(jnp.where(iota16 == 0, rcs, 0))
        return wr + tot

    lax.fori_loop(0, _TVEC, _pbody, jnp.zeros((16,), jnp.int32))

    slot = pl.multiple_of(tid * _CAP_T, 128)
    pltpu.sync_copy(idxbuf.at[pl.ds(0, _CAP_T)], idx_hbm.at[pl.ds(slot, _CAP_T)])

    pend = pltpu.async_copy(planes[0].at[pl.ds(base_word, _TW)],
                            pbuf.at[pl.ds(0, _TW)], sem)
    for c in range(_C):
        cur = pend
        if c + 1 < _C:
            pend = pltpu.async_copy(
                planes[c + 1].at[pl.ds(base_word, _TW)],
                pbuf.at[pl.ds(((c + 1) % 2) * _TW, _TW)], sem)
        cur.wait()
        for g in range(_CAP_T // 16):
            pv = idxbuf[pl.ds(g * 16, 16)]
            lp = jnp.clip(pv - base_word, 0, _TW - 1)
            sbuf[pl.ds(g * 16, 16)] = plsc.load_gather(pbuf.at[pl.ds((c % 2) * _TW, _TW)], [lp])
        pltpu.sync_copy(
            sbuf, lcomp_hbm.at[pl.ds(
                pl.multiple_of(c * _CAPE + slot, 128), _CAP_T)])


def _stage2(mask, planes):
    f = functools.partial(
        pl.kernel,
        out_type=[
            jax.ShapeDtypeStruct((_CAPE,), jnp.int32),
            jax.ShapeDtypeStruct((_C * _CAPE,), jnp.float32),
        ],
        mesh=plsc.VectorSubcoreMesh(core_axis_name="c", subcore_axis_name="s"),
        compiler_params=pltpu.CompilerParams(needs_layout_passes=False),
        scratch_types=[
            pltpu.VMEM((_TW,), jnp.int32),
            pltpu.VMEM((_CAP_T + 128,), jnp.int32),
            pltpu.VMEM((2 * _TW,), jnp.float32),
            pltpu.VMEM((_CAP_T,), jnp.float32),
            pltpu.SemaphoreType.DMA,
        ],
    )(_s2_body)
    return f(mask, *planes)


def _s3_body(idx_hbm, lcomp_hbm, samp_hbm, idxr_hbm, vidx, vpl, vsamp, vidxr,
             sem1, sem2, sem3, sem4):
    def _body(j, _):
        row = j * 16
        cp = pltpu.make_async_copy(
            idx_hbm.at[(pl.ds(row, 16), slice(None))], vidx, sem1)
        cp.start()
        descs = [
            pltpu.async_copy(
                lcomp_hbm.at[(pl.ds(c * _CAPL + row, 16), slice(None))],
                vpl.at[pl.ds(c * 16, 16)], sem2)
            for c in range(_C)
        ]
        cp.wait()
        rp = vidx[...]
        tp = jnp.where(rp < _SENT, (rp % _N) * _B + rp // _N, rp)
        vidxr[...] = tp
        p = tp.astype(jnp.uint32) << 5
        for d in descs:
            d.wait()
        planes = vpl[...]
        best = _gumbel_from_counts(p) + planes[0:16, :]
        bi = jnp.zeros((16, 128), jnp.int32)
        for c in range(1, _C):
            sc = (_gumbel_from_counts(p + jnp.uint32(c))
                  + planes[c * 16:(c + 1) * 16, :])
            upd = sc > best
            best = jnp.where(upd, sc, best)
            bi = jnp.where(upd, c, bi)
        vsamp[...] = bi
        cpo = pltpu.make_async_copy(
            vsamp, samp_hbm.at[(pl.ds(row, 16), slice(None))], sem3)
        cpo.start()
        cpi = pltpu.make_async_copy(
            vidxr, idxr_hbm.at[(pl.ds(row, 16), slice(None))], sem4)
        cpi.start()
        cpo.wait()
        cpi.wait()
        return 0

    lax.fori_loop(0, _CAPL // 16, _body, jnp.int32(0))


def _stage3(idx2, lcomp2):
    return pl.pallas_call(
        _s3_body,
        in_specs=[
            pl.BlockSpec(memory_space=pltpu.MemorySpace.HBM),
            pl.BlockSpec(memory_space=pltpu.MemorySpace.HBM),
        ],
        out_specs=[pl.BlockSpec(memory_space=pltpu.MemorySpace.HBM)] * 2,
        out_shape=[jax.ShapeDtypeStruct((_CAPL, 128), jnp.int32)] * 2,
        scratch_shapes=[
            pltpu.VMEM((16, 128), jnp.int32),
            pltpu.VMEM((16 * _C, 128), jnp.float32),
            pltpu.VMEM((16, 128), jnp.int32),
            pltpu.VMEM((16, 128), jnp.int32),
            pltpu.SemaphoreType.DMA,
            pltpu.SemaphoreType.DMA,
            pltpu.SemaphoreType.DMA,
            pltpu.SemaphoreType.DMA,
        ],
    )(idx2, lcomp2)


_ZW = _AEXT // 16
_S4CH = 2048


def _s4_body(idx_hbm, samp_hbm, act_hbm, abuf, ibuf, vbuf, sem):
    sid = lax.axis_index("s")
    iota16 = lax.iota(jnp.int32, 16)
    seg_lo = sid * _ZW

    def _zb(k, _):
        abuf[pl.ds(k * 16, 16)] = jnp.zeros((16,), jnp.int32)
        return 0

    lax.fori_loop(0, _ZW // 16, _zb, jnp.int32(0))

    def _chunk(q, _):
        off = pl.multiple_of(q * _S4CH, 8)
        pltpu.sync_copy(idx_hbm.at[pl.ds(off, _S4CH)], ibuf)
        pltpu.sync_copy(samp_hbm.at[pl.ds(off, _S4CH)], vbuf)
        for g in range(_S4CH // 16):
            iv = ibuf[pl.ds(g * 16, 16)]
            sv = vbuf[pl.ds(g * 16, 16)]
            loc = iv - seg_lo
            m = (loc >= 0) & (loc < _ZW)
            plsc.store_scatter(abuf, [jnp.clip(loc, 0, _ZW - 1)], sv, mask=m)
        return 0

    lax.fori_loop(0, _CAPE // _S4CH, _chunk, jnp.int32(0))
    pltpu.sync_copy(abuf,
                    act_hbm.at[pl.ds(pl.multiple_of(seg_lo, 8), _ZW)])


def _stage4(idxc, samp):
    f = functools.partial(
        pl.kernel,
        out_type=jax.ShapeDtypeStruct((_AEXT,), jnp.int32),
        mesh=plsc.VectorSubcoreMesh(core_axis_name="c", subcore_axis_name="s",
                                    num_cores=1),
        compiler_params=pltpu.CompilerParams(needs_layout_passes=False),
        scratch_types=[
            pltpu.VMEM((_ZW,), jnp.int32),
            pltpu.VMEM((_S4CH,), jnp.int32),
            pltpu.VMEM((_S4CH,), jnp.int32),
            pltpu.SemaphoreType.DMA,
        ],
    )(_s4_body)
    return f(idxc, samp)


def kernel(ob, edge_index, W1, b1, W2, b2):
    del edge_index
    obp = jnp.transpose(ob, (2, 1, 0)).reshape(_C + 2, _ROWS)
    mask, *planes = _stage1(obp, W1, b1, W2, b2)
    tile_counts = jnp.sum(mask.reshape(32, _TW), axis=1)
    overflow = jnp.any(tile_counts > _CAP_T)

    def _sparse(_):
        idxc, lcomp = _stage2(mask, planes)
        idx2 = idxc.reshape(_CAPL, 128)
        lcomp2 = lcomp.reshape(_C * _CAPL, 128)
        samp, idxr = _stage3(idx2, lcomp2)
        act = _stage4(idxr.reshape(_CAPE), samp.reshape(_CAPE))
        return act[:_ROWS].reshape(_N, _B)

    def _dense(_):
        return _dense_kernel(ob, W1, b1, W2, b2)

    return lax.cond(overflow, _dense, _sparse, None)
